# Initial kernel scaffold; baseline (speedup 1.0000x reference)
import jax
import jax.numpy as jnp
from jax.experimental import pallas as pl


def kernel(X, Z):
    raise NotImplementedError("write your pallas kernel here")


# speedup vs baseline: 3.3835x; 3.3835x over previous
import math

import numpy as np

import jax
import jax.numpy as jnp
from jax import lax
from jax.experimental import pallas as pl
from jax.experimental.pallas import tpu as pltpu

_LOG2E = 1.4426950408889634
_BIG = 1e30


def _round_up(x, m):
    return ((x + m - 1) // m) * m


def _pick_tile(n_pad):
    for t in (1024, 512, 256, 128):
        if n_pad % t == 0:
            return t
    return 128


def _sym_tile_kernel(ii_ref, jj_ref, a_ref, b_ref, qn_ref, o_ref):
    s0 = pl.program_id(0)
    s1 = pl.program_id(1)
    i = ii_ref[s0, s1]
    j = jj_ref[s0, s1]

    @pl.when(j == i)
    def _init():
        o_ref[...] = jnp.zeros_like(o_ref)

    dots = lax.dot_general(
        a_ref[...], b_ref[...], (((1,), (1,)), ((), ())),
        preferred_element_type=jnp.float32,
    )
    e = jnp.exp2(dots - qn_ref[0])
    rows = jnp.sum(e, axis=-1, keepdims=True)
    w = jnp.where(j > i, 2.0, 1.0).astype(jnp.float32)
    o_ref[...] = o_ref[...] + rows * w


def _cross_tile_kernel(a_ref, b_ref, qn_ref, o_ref):
    j = pl.program_id(1)

    @pl.when(j == 0)
    def _init():
        o_ref[...] = jnp.zeros_like(o_ref)

    dots = lax.dot_general(
        a_ref[...], b_ref[...], (((1,), (1,)), ((), ())),
        preferred_element_type=jnp.float32,
    )
    e = jnp.exp2(dots - qn_ref[0])
    o_ref[...] = o_ref[...] + jnp.sum(e, axis=-1, keepdims=True)


def _sym_rowsums(a_scaled, b_plain, qn, T, nt):
    n_pad, D = a_scaled.shape
    if nt % 2 == 0 and nt > 1:
        g0, g1 = nt // 2, nt + 1
        ii = np.zeros((g0, g1), np.int32)
        jj = np.zeros((g0, g1), np.int32)
        for s0 in range(g0):
            r0, r1 = s0, nt - 1 - s0
            tiles = [(r0, j) for j in range(r0, nt)]
            tiles += [(r1, j) for j in range(r1, nt)]
            for s1, (ti, tj) in enumerate(tiles):
                ii[s0, s1], jj[s0, s1] = ti, tj
    else:
        tri = [(i, j) for i in range(nt) for j in range(i, nt)]
        g0, g1 = 1, len(tri)
        ii = np.asarray([t[0] for t in tri], np.int32).reshape(1, -1)
        jj = np.asarray([t[1] for t in tri], np.int32).reshape(1, -1)

    out = pl.pallas_call(
        _sym_tile_kernel,
        out_shape=jax.ShapeDtypeStruct((n_pad, 128), jnp.float32),
        grid_spec=pltpu.PrefetchScalarGridSpec(
            num_scalar_prefetch=2,
            grid=(g0, g1),
            in_specs=[
                pl.BlockSpec((T, D), lambda s0, s1, ii, jj: (ii[s0, s1], 0)),
                pl.BlockSpec((T, D), lambda s0, s1, ii, jj: (jj[s0, s1], 0)),
                pl.BlockSpec((1, 1, T), lambda s0, s1, ii, jj: (jj[s0, s1], 0, 0)),
            ],
            out_specs=pl.BlockSpec((T, 128), lambda s0, s1, ii, jj: (ii[s0, s1], 0)),
        ),
        compiler_params=pltpu.CompilerParams(
            dimension_semantics=("arbitrary", "arbitrary"),
            vmem_limit_bytes=100 * 1024 * 1024,
        ),
    )(jnp.asarray(ii), jnp.asarray(jj), a_scaled, b_plain, qn)
    return out[:, 0]


def _cross_rowsums(a_scaled, b_plain, qn_b, TM, TN):
    n_pad, D = a_scaled.shape
    m_pad, _ = b_plain.shape
    out = pl.pallas_call(
        _cross_tile_kernel,
        out_shape=jax.ShapeDtypeStruct((n_pad, 128), jnp.float32),
        grid=(n_pad // TM, m_pad // TN),
        in_specs=[
            pl.BlockSpec((TM, D), lambda i, j: (i, 0)),
            pl.BlockSpec((TN, D), lambda i, j: (j, 0)),
            pl.BlockSpec((1, 1, TN), lambda i, j: (j, 0, 0)),
        ],
        out_specs=pl.BlockSpec((TM, 128), lambda i, j: (i, 0)),
        compiler_params=pltpu.CompilerParams(
            dimension_semantics=("arbitrary", "arbitrary"),
            vmem_limit_bytes=100 * 1024 * 1024,
        ),
    )(a_scaled, b_plain, qn_b)
    return out[:, 0]


def _prep(P, ksize):
    n, d = P.shape
    P32 = P.astype(jnp.float32)
    q = _LOG2E / float(ksize)
    n_pad = _round_up(n, 128)
    T = _pick_tile(n_pad)
    n_pad = _round_up(n, T)
    nt = n_pad // T

    if n_pad != n:
        P32 = jnp.zeros((n_pad, d), jnp.float32).at[:n].set(P32)
    scaled = (P32 * (2.0 * q)).astype(jnp.bfloat16)
    plain = P32.astype(jnp.bfloat16)
    qn = jnp.sum(P32 * P32, axis=-1) * q
    if n_pad != n:
        mask = jnp.arange(n_pad) < n
        qn = jnp.where(mask, qn, _BIG)
    rowfac = jnp.exp2(-qn)
    qn_rows = qn.reshape(nt, 1, T)
    return scaled, plain, qn_rows, rowfac, T, nt


def kernel(X, Z):
    ksize = 64.0
    N, D = X.shape
    M, D2 = Z.shape
    assert D == D2, "feature dims must match"
    norm = math.sqrt(2.0 * math.pi * ksize)

    Xs, Xp, qn_x, fac_x, Tx, ntx = _prep(X, ksize)
    Zs, Zp, qn_z, fac_z, Tz, ntz = _prep(Z, ksize)

    rs_xx = _sym_rowsums(Xs, Xp, qn_x, Tx, ntx)
    rs_zz = _sym_rowsums(Zs, Zp, qn_z, Tz, ntz)
    rs_xz = _cross_rowsums(Xs, Zp, qn_z, Tx, Tz)

    s_xx = jnp.sum(fac_x * rs_xx)
    s_zz = jnp.sum(fac_z * rs_zz)
    s_xz = jnp.sum(fac_x * rs_xz)

    m_xx = s_xx / (norm * N * N)
    m_zz = s_zz / (norm * M * M)
    m_xz = s_xz / (norm * N * M)
    return jnp.log(jnp.sqrt(m_xx * m_zz + 1e-5) / (m_xz + 1e-5))


# speedup vs baseline: 4.8850x; 1.4438x over previous
import math

import numpy as np

import jax
import jax.numpy as jnp
from jax import lax
from jax.experimental import pallas as pl
from jax.experimental.pallas import tpu as pltpu

_LOG2E = 1.4426950408889634
_BIG = 1e30


def _round_up(x, m):
    return ((x + m - 1) // m) * m


def _pick_tile(n_pad):
    for t in (2048, 1024, 512, 256, 128):
        if n_pad % t == 0:
            return t
    return 128


def _sym_tile_kernel(ii_ref, jj_ref, a_ref, b_ref, qn_ref, o_ref):
    s0 = pl.program_id(0)
    s1 = pl.program_id(1)
    i = ii_ref[s0, s1]
    j = jj_ref[s0, s1]

    @pl.when(j == i)
    def _init():
        o_ref[...] = jnp.zeros_like(o_ref)

    dots = lax.dot_general(
        a_ref[...], b_ref[...], (((1,), (1,)), ((), ())),
        preferred_element_type=jnp.float32,
    )
    e = jnp.exp2(dots - qn_ref[0])
    rows = jnp.sum(e, axis=-1, keepdims=True)
    w = jnp.where(j > i, 2.0, 1.0).astype(jnp.float32)
    o_ref[...] = o_ref[...] + rows * w


def _cross_tile_kernel(a_ref, b_ref, qn_ref, o_ref):
    j = pl.program_id(1)

    @pl.when(j == 0)
    def _init():
        o_ref[...] = jnp.zeros_like(o_ref)

    dots = lax.dot_general(
        a_ref[...], b_ref[...], (((1,), (1,)), ((), ())),
        preferred_element_type=jnp.float32,
    )
    e = jnp.exp2(dots - qn_ref[0])
    o_ref[...] = o_ref[...] + jnp.sum(e, axis=-1, keepdims=True)


def _sym_rowsums(a_scaled, b_plain, qn, T, nt):
    n_pad, D = a_scaled.shape
    if nt % 2 == 0 and nt > 1:
        g0, g1 = nt // 2, nt + 1
        ii = np.zeros((g0, g1), np.int32)
        jj = np.zeros((g0, g1), np.int32)
        for s0 in range(g0):
            r0, r1 = s0, nt - 1 - s0
            tiles = [(r0, j) for j in range(r0, nt)]
            tiles += [(r1, j) for j in range(r1, nt)]
            for s1, (ti, tj) in enumerate(tiles):
                ii[s0, s1], jj[s0, s1] = ti, tj
    else:
        tri = [(i, j) for i in range(nt) for j in range(i, nt)]
        g0, g1 = 1, len(tri)
        ii = np.asarray([t[0] for t in tri], np.int32).reshape(1, -1)
        jj = np.asarray([t[1] for t in tri], np.int32).reshape(1, -1)

    out = pl.pallas_call(
        _sym_tile_kernel,
        out_shape=jax.ShapeDtypeStruct((n_pad, 128), jnp.float32),
        grid_spec=pltpu.PrefetchScalarGridSpec(
            num_scalar_prefetch=2,
            grid=(g0, g1),
            in_specs=[
                pl.BlockSpec((T, D), lambda s0, s1, ii, jj: (ii[s0, s1], 0)),
                pl.BlockSpec((T, D), lambda s0, s1, ii, jj: (jj[s0, s1], 0)),
                pl.BlockSpec((1, 1, T), lambda s0, s1, ii, jj: (jj[s0, s1], 0, 0)),
            ],
            out_specs=pl.BlockSpec((T, 128), lambda s0, s1, ii, jj: (ii[s0, s1], 0)),
        ),
        compiler_params=pltpu.CompilerParams(
            dimension_semantics=("arbitrary", "arbitrary"),
            vmem_limit_bytes=100 * 1024 * 1024,
        ),
    )(jnp.asarray(ii), jnp.asarray(jj), a_scaled, b_plain, qn)
    return out[:, 0]


def _cross_rowsums(a_scaled, b_plain, qn_b, TM, TN):
    n_pad, D = a_scaled.shape
    m_pad, _ = b_plain.shape
    out = pl.pallas_call(
        _cross_tile_kernel,
        out_shape=jax.ShapeDtypeStruct((n_pad, 128), jnp.float32),
        grid=(n_pad // TM, m_pad // TN),
        in_specs=[
            pl.BlockSpec((TM, D), lambda i, j: (i, 0)),
            pl.BlockSpec((TN, D), lambda i, j: (j, 0)),
            pl.BlockSpec((1, 1, TN), lambda i, j: (j, 0, 0)),
        ],
        out_specs=pl.BlockSpec((TM, 128), lambda i, j: (i, 0)),
        compiler_params=pltpu.CompilerParams(
            dimension_semantics=("arbitrary", "arbitrary"),
            vmem_limit_bytes=100 * 1024 * 1024,
        ),
    )(a_scaled, b_plain, qn_b)
    return out[:, 0]


def _prep(P, ksize):
    n, d = P.shape
    P32 = P.astype(jnp.float32)
    q = _LOG2E / float(ksize)
    n_pad = _round_up(n, 128)
    T = _pick_tile(n_pad)
    n_pad = _round_up(n, T)
    nt = n_pad // T

    if n_pad != n:
        P32 = jnp.zeros((n_pad, d), jnp.float32).at[:n].set(P32)
    scaled = (P32 * (2.0 * q)).astype(jnp.bfloat16)
    plain = P32.astype(jnp.bfloat16)
    qn = jnp.sum(P32 * P32, axis=-1) * q
    if n_pad != n:
        mask = jnp.arange(n_pad) < n
        qn = jnp.where(mask, qn, _BIG)
    rowfac = jnp.exp2(-qn)
    qn_rows = qn.reshape(nt, 1, T)
    return scaled, plain, qn_rows, rowfac, T, nt


def kernel(X, Z):
    ksize = 64.0
    N, D = X.shape
    M, D2 = Z.shape
    assert D == D2, "feature dims must match"
    norm = math.sqrt(2.0 * math.pi * ksize)

    Xs, Xp, qn_x, fac_x, Tx, ntx = _prep(X, ksize)
    Zs, Zp, qn_z, fac_z, Tz, ntz = _prep(Z, ksize)

    rs_xx = _sym_rowsums(Xs, Xp, qn_x, Tx, ntx)
    rs_zz = _sym_rowsums(Zs, Zp, qn_z, Tz, ntz)
    rs_xz = _cross_rowsums(Xs, Zp, qn_z, Tx, Tz)

    s_xx = jnp.sum(fac_x * rs_xx)
    s_zz = jnp.sum(fac_z * rs_zz)
    s_xz = jnp.sum(fac_x * rs_xz)

    m_xx = s_xx / (norm * N * N)
    m_zz = s_zz / (norm * M * M)
    m_xz = s_xz / (norm * N * M)
    return jnp.log(jnp.sqrt(m_xx * m_zz + 1e-5) / (m_xz + 1e-5))
